# baseline (speedup 1.0000x reference)
import jax
import jax.numpy as jnp
from jax.experimental import pallas as pl

CONTEXT = 16
VOCAB = 256
EMBED = 6
HIDDEN = 64
NOUT = 256
BB = 512


def _table_body(embed_ref, w1r_ref, b1_ref, tbl_ref):
    tbl_ref[0] = (
        jnp.dot(embed_ref[...], w1r_ref[0], preferred_element_type=jnp.float32)
        + b1_ref[...] / CONTEXT
    ).astype(jnp.bfloat16)


def _mlp_body(x_ref, tbl_ref, w2t_ref, b2_ref, out_ref):
    acc = jnp.zeros((BB, HIDDEN), dtype=jnp.float32)
    iota = jax.lax.broadcasted_iota(jnp.int32, (BB, VOCAB), 1)
    for t in range(CONTEXT):
        col = x_ref[:, t : t + 1]
        mh = (col == iota).astype(jnp.bfloat16)
        acc = acc + jnp.dot(mh, tbl_ref[t], preferred_element_type=jnp.float32)
    h1 = jnp.maximum(acc, 0.0).astype(jnp.bfloat16)
    out = jnp.dot(h1, w2t_ref[...], preferred_element_type=jnp.float32)
    out_ref[...] = out + b2_ref[...]


def kernel(x, embed, W1, b1, W2, b2):
    batch = x.shape[0]
    x = x.astype(jnp.int32)
    w1r = W1.reshape(HIDDEN, CONTEXT, EMBED).transpose(1, 2, 0)
    b1_2d = b1.reshape(1, HIDDEN)
    w2t = W2.T
    b2_2d = b2.reshape(1, NOUT)

    tbl = pl.pallas_call(
        _table_body,
        grid=(CONTEXT,),
        in_specs=[
            pl.BlockSpec((VOCAB, EMBED), lambda t: (0, 0)),
            pl.BlockSpec((1, EMBED, HIDDEN), lambda t: (t, 0, 0)),
            pl.BlockSpec((1, HIDDEN), lambda t: (0, 0)),
        ],
        out_specs=pl.BlockSpec((1, VOCAB, HIDDEN), lambda t: (t, 0, 0)),
        out_shape=jax.ShapeDtypeStruct((CONTEXT, VOCAB, HIDDEN), jnp.bfloat16),
    )(embed, w1r, b1_2d)
    w2t = w2t.astype(jnp.bfloat16)

    out = pl.pallas_call(
        _mlp_body,
        grid=(batch // BB,),
        in_specs=[
            pl.BlockSpec((BB, CONTEXT), lambda i: (i, 0)),
            pl.BlockSpec((CONTEXT, VOCAB, HIDDEN), lambda i: (0, 0, 0)),
            pl.BlockSpec((HIDDEN, NOUT), lambda i: (0, 0)),
            pl.BlockSpec((1, NOUT), lambda i: (0, 0)),
        ],
        out_specs=pl.BlockSpec((BB, NOUT), lambda i: (i, 0)),
        out_shape=jax.ShapeDtypeStruct((batch, NOUT), jnp.float32),
    )(x, tbl, w2t, b2_2d)
    return out


# speedup vs baseline: 1.1560x; 1.1560x over previous
import dataclasses
import functools

import jax
import jax.numpy as jnp
from jax import lax
from jax.experimental import pallas as pl
from jax.experimental.pallas import tpu as pltpu
from jax.experimental.pallas import tpu_sc as plsc

CONTEXT = 16
VOCAB = 256
EMBED = 6
HIDDEN = 64
NOUT = 256
NFEAT = CONTEXT * EMBED

NUM_CORES = 2
NUM_SUBCORES = 16
NW = NUM_CORES * NUM_SUBCORES
LANES = 16


def _sc_gather_body(emb_hbm, xprep_hbm, out_hbm, emb_v, xv, h0t_v, sem):
    bpw = h0t_v.shape[1]
    wid = lax.axis_index("s") * NUM_CORES + lax.axis_index("c")
    pltpu.sync_copy(emb_hbm, emb_v)
    pltpu.sync_copy(xprep_hbm.at[pl.ds(wid * bpw * CONTEXT, bpw * CONTEXT)], xv)

    @pl.loop(0, bpw, step=LANES)
    def _(b):
        for t in range(CONTEXT):
            xvals = xv[pl.ds(t * bpw + b, LANES)]
            addr = xvals * EMBED
            for d in range(EMBED):
                v = plsc.load_gather(emb_v, [addr + d])
                h0t_v[t * EMBED + d, pl.ds(b, LANES)] = v

    pltpu.async_copy(h0t_v, out_hbm.at[wid], sem).wait()


def _mlp_body(h0t_ref, w1_ref, b1_ref, w2_ref, b2_ref, out_ref):
    h0t = h0t_ref[0].astype(jnp.bfloat16)
    h1t = lax.dot_general(
        w1_ref[...], h0t, (((1,), (0,)), ((), ())),
        preferred_element_type=jnp.float32,
    )
    h1t = jnp.maximum(h1t + b1_ref[...], 0.0).astype(jnp.bfloat16)
    outt = lax.dot_general(
        w2_ref[...], h1t, (((1,), (0,)), ((), ())),
        preferred_element_type=jnp.float32,
    )
    out_ref[...] = (outt + b2_ref[...]).T


def kernel(x, embed, W1, b1, W2, b2):
    batch = x.shape[0]
    bpw = batch // NW
    x = x.astype(jnp.int32)
    xprep = x.reshape(NW, bpw, CONTEXT).transpose(0, 2, 1).reshape(-1)
    emb_flat = embed.reshape(VOCAB * EMBED)

    cp = pltpu.CompilerParams()
    if "needs_layout_passes" in pltpu.CompilerParams.__dataclass_fields__:
        cp = dataclasses.replace(cp, needs_layout_passes=False)
    mesh = plsc.VectorSubcoreMesh(core_axis_name="c", subcore_axis_name="s")
    sc_gather = functools.partial(
        pl.kernel,
        mesh=mesh,
        compiler_params=cp,
        out_type=jax.ShapeDtypeStruct((NW, NFEAT, bpw), jnp.float32),
        scratch_types=[
            pltpu.VMEM((VOCAB * EMBED,), jnp.float32),
            pltpu.VMEM((bpw * CONTEXT,), jnp.int32),
            pltpu.VMEM((NFEAT, bpw), jnp.float32),
            pltpu.SemaphoreType.DMA,
        ],
    )(_sc_gather_body)
    h0t = sc_gather(emb_flat, xprep)

    w1_bf = W1.astype(jnp.bfloat16)
    w2_bf = W2.astype(jnp.bfloat16)
    b1_col = b1.reshape(HIDDEN, 1)
    b2_col = b2.reshape(NOUT, 1)

    out = pl.pallas_call(
        _mlp_body,
        grid=(NW,),
        in_specs=[
            pl.BlockSpec((1, NFEAT, bpw), lambda i: (i, 0, 0)),
            pl.BlockSpec((HIDDEN, NFEAT), lambda i: (0, 0)),
            pl.BlockSpec((HIDDEN, 1), lambda i: (0, 0)),
            pl.BlockSpec((NOUT, HIDDEN), lambda i: (0, 0)),
            pl.BlockSpec((NOUT, 1), lambda i: (0, 0)),
        ],
        out_specs=pl.BlockSpec((bpw, NOUT), lambda i: (i, 0)),
        out_shape=jax.ShapeDtypeStruct((batch, NOUT), jnp.float32),
    )(h0t, w1_bf, b1_col, w2_bf, b2_col)
    return out


# speedup vs baseline: 1.2946x; 1.1200x over previous
import dataclasses
import functools

import jax
import jax.numpy as jnp
from jax import lax
from jax.experimental import pallas as pl
from jax.experimental.pallas import tpu as pltpu
from jax.experimental.pallas import tpu_sc as plsc

CONTEXT = 16
VOCAB = 256
EMBED = 6
HIDDEN = 64
NOUT = 256
NFEAT = CONTEXT * EMBED

NUM_CORES = 2
NUM_SUBCORES = 16
NW = NUM_CORES * NUM_SUBCORES
LANES = 16


def _sc_gather_body(emb_hbm, xprep_hbm, out_hbm, emb_v, xv, h0t_v, sem):
    bpw = h0t_v.shape[1]
    wid = lax.axis_index("s") * NUM_CORES + lax.axis_index("c")
    pltpu.sync_copy(emb_hbm, emb_v)
    pltpu.sync_copy(xprep_hbm.at[pl.ds(wid * bpw * CONTEXT, bpw * CONTEXT)], xv)

    @plsc.parallel_loop(0, bpw, step=LANES, unroll=4)
    def _(b):
        for t in range(CONTEXT):
            addr = xv[pl.ds(t * bpw + b, LANES)]
            for d in range(EMBED):
                v = plsc.load_gather(emb_v, [addr + d] if d else [addr])
                h0t_v[t * EMBED + d, pl.ds(b, LANES)] = v

    pltpu.async_copy(h0t_v, out_hbm.at[wid], sem).wait()


def _mlp_body(h0t_ref, w1_ref, b1_ref, w2_ref, b2_ref, out_ref):
    h0t = h0t_ref[0].astype(jnp.bfloat16)
    h1t = lax.dot_general(
        w1_ref[...], h0t, (((1,), (0,)), ((), ())),
        preferred_element_type=jnp.float32,
    )
    h1t = jnp.maximum(h1t + b1_ref[...], 0.0).astype(jnp.bfloat16)
    outt = lax.dot_general(
        w2_ref[...], h1t, (((1,), (0,)), ((), ())),
        preferred_element_type=jnp.float32,
    )
    out_ref[...] = (outt + b2_ref[...]).T


def kernel(x, embed, W1, b1, W2, b2):
    batch = x.shape[0]
    bpw = batch // NW
    x = x.astype(jnp.int32)
    xprep = (x * EMBED).reshape(NW, bpw, CONTEXT).transpose(0, 2, 1).reshape(-1)
    emb_flat = embed.reshape(VOCAB * EMBED)

    cp = pltpu.CompilerParams()
    if "needs_layout_passes" in pltpu.CompilerParams.__dataclass_fields__:
        cp = dataclasses.replace(cp, needs_layout_passes=False)
    mesh = plsc.VectorSubcoreMesh(core_axis_name="c", subcore_axis_name="s")
    sc_gather = functools.partial(
        pl.kernel,
        mesh=mesh,
        compiler_params=cp,
        out_type=jax.ShapeDtypeStruct((NW, NFEAT, bpw), jnp.float32),
        scratch_types=[
            pltpu.VMEM((VOCAB * EMBED,), jnp.float32),
            pltpu.VMEM((bpw * CONTEXT,), jnp.int32),
            pltpu.VMEM((NFEAT, bpw), jnp.float32),
            pltpu.SemaphoreType.DMA,
        ],
    )(_sc_gather_body)
    h0t = sc_gather(emb_flat, xprep)

    w1_bf = W1.astype(jnp.bfloat16)
    w2_bf = W2.astype(jnp.bfloat16)
    b1_col = b1.reshape(HIDDEN, 1)
    b2_col = b2.reshape(NOUT, 1)

    out = pl.pallas_call(
        _mlp_body,
        grid=(NW,),
        in_specs=[
            pl.BlockSpec((1, NFEAT, bpw), lambda i: (i, 0, 0)),
            pl.BlockSpec((HIDDEN, NFEAT), lambda i: (0, 0)),
            pl.BlockSpec((HIDDEN, 1), lambda i: (0, 0)),
            pl.BlockSpec((NOUT, HIDDEN), lambda i: (0, 0)),
            pl.BlockSpec((NOUT, 1), lambda i: (0, 0)),
        ],
        out_specs=pl.BlockSpec((bpw, NOUT), lambda i: (i, 0)),
        out_shape=jax.ShapeDtypeStruct((batch, NOUT), jnp.float32),
    )(h0t, w1_bf, b1_col, w2_bf, b2_col)
    return out
